# Initial kernel scaffold; baseline (speedup 1.0000x reference)
import jax
import jax.numpy as jnp
from jax.experimental import pallas as pl


def kernel(x, table):
    raise NotImplementedError("write your pallas kernel here")


# speedup vs baseline: 3.1942x; 3.1942x over previous
import functools
import math

import jax
import jax.numpy as jnp
import numpy as np
from jax import lax
from jax.experimental import pallas as pl
from jax.experimental.pallas import tpu as pltpu
from jax.experimental.pallas import tpu_sc as plsc

D_MODEL = 64
MAX_LEN = 5000
BATCH = 4096
SEQ = 200
SCALE = math.sqrt(D_MODEL)

NC, NS = 2, 16
NW = NC * NS
ROWS = BATCH * SEQ
SEQ_PER_W = BATCH // NW
ROWS_PER_W = SEQ_PER_W * SEQ

IDX_COLS = 100
IDX_ROWS_PER_SEQ = SEQ // IDX_COLS


def _make_pe():
    pe = np.zeros((MAX_LEN, D_MODEL), dtype=np.float32)
    pos = np.arange(MAX_LEN, dtype=np.float32)[:, None]
    div_term = np.exp(
        np.arange(0, D_MODEL, 2, dtype=np.float32) * (-math.log(10000.0) / D_MODEL)
    )
    pe[:, 0::2] = np.sin(pos * div_term)
    pe[:, 1::2] = np.cos(pos * div_term)
    return jnp.asarray(pe[:SEQ])


_mesh = plsc.VectorSubcoreMesh(core_axis_name="c", subcore_axis_name="s")


@functools.partial(
    pl.kernel,
    out_type=jax.ShapeDtypeStruct((ROWS, D_MODEL), jnp.float32),
    mesh=_mesh,
    scratch_types=[
        pltpu.VMEM((SEQ_PER_W * IDX_ROWS_PER_SEQ, IDX_COLS), jnp.int32),
        pltpu.VMEM((SEQ, D_MODEL), jnp.float32),
        pltpu.VMEM((SEQ, D_MODEL), jnp.float32),
        pltpu.SemaphoreType.DMA,
    ],
    compiler_params=pltpu.CompilerParams(use_tc_tiling_on_sc=False),
)
def _sc_kernel(x_hbm, pe_hbm, table_hbm, out_hbm, idx_v, pe_v, rows_v, sem):
    wid = lax.axis_index("s") * NC + lax.axis_index("c")
    idx_base = wid * SEQ_PER_W * IDX_ROWS_PER_SEQ
    row_base = wid * ROWS_PER_W

    pltpu.sync_copy(x_hbm.at[pl.ds(idx_base, SEQ_PER_W * IDX_ROWS_PER_SEQ)], idx_v)
    pltpu.sync_copy(pe_hbm, pe_v)

    def step(t, _):
        g0 = pltpu.async_copy(
            table_hbm.at[idx_v.at[2 * t]], rows_v.at[pl.ds(0, IDX_COLS)], sem
        )
        g1 = pltpu.async_copy(
            table_hbm.at[idx_v.at[2 * t + 1]], rows_v.at[pl.ds(IDX_COLS, IDX_COLS)], sem
        )
        g0.wait()
        g1.wait()

        def fma_row(r, _):
            for j in range(D_MODEL // 16):
                sl = pl.ds(j * 16, 16)
                rows_v[r, sl] = rows_v[r, sl] * SCALE + pe_v[r, sl]
            return ()

        lax.fori_loop(0, SEQ, fma_row, ())
        pltpu.sync_copy(rows_v, out_hbm.at[pl.ds(row_base + t * SEQ, SEQ)])
        return ()

    lax.fori_loop(0, SEQ_PER_W, step, ())


def kernel(x, table):
    pe = _make_pe()
    x2 = x.reshape(2 * BATCH, IDX_COLS)
    out = _sc_kernel(x2, pe, table)
    return out.reshape(BATCH, SEQ, D_MODEL)


# speedup vs baseline: 4.1941x; 1.3131x over previous
import functools
import math

import jax
import jax.numpy as jnp
import numpy as np
from jax import lax
from jax.experimental import pallas as pl
from jax.experimental.pallas import tpu as pltpu
from jax.experimental.pallas import tpu_sc as plsc

D_MODEL = 64
MAX_LEN = 5000
BATCH = 4096
SEQ = 200
SCALE = math.sqrt(D_MODEL)

NC, NS = 2, 16
NW = NC * NS
ROWS = BATCH * SEQ
SEQ_PER_W = BATCH // NW
ROWS_PER_W = SEQ_PER_W * SEQ

IDX_COLS = 100
IDX_ROWS_PER_SEQ = SEQ // IDX_COLS

NBUF = 4
LOOKAHEAD = 2


def _make_pe():
    pe = np.zeros((MAX_LEN, D_MODEL), dtype=np.float32)
    pos = np.arange(MAX_LEN, dtype=np.float32)[:, None]
    div_term = np.exp(
        np.arange(0, D_MODEL, 2, dtype=np.float32) * (-math.log(10000.0) / D_MODEL)
    )
    pe[:, 0::2] = np.sin(pos * div_term)
    pe[:, 1::2] = np.cos(pos * div_term)
    return jnp.asarray(pe[:SEQ])


_mesh = plsc.VectorSubcoreMesh(core_axis_name="c", subcore_axis_name="s")


@functools.partial(
    pl.kernel,
    out_type=jax.ShapeDtypeStruct((ROWS, D_MODEL), jnp.float32),
    mesh=_mesh,
    scratch_types=[
        pltpu.VMEM((SEQ_PER_W * IDX_ROWS_PER_SEQ, IDX_COLS), jnp.int32),
        pltpu.VMEM((SEQ, D_MODEL), jnp.float32),
        [pltpu.VMEM((SEQ, D_MODEL), jnp.float32) for _ in range(NBUF)],
        [pltpu.SemaphoreType.DMA for _ in range(NBUF)],
        [pltpu.SemaphoreType.DMA for _ in range(NBUF)],
    ],
    compiler_params=pltpu.CompilerParams(use_tc_tiling_on_sc=False),
)
def _sc_kernel(x_hbm, pe_hbm, table_hbm, out_hbm, idx_v, pe_v, rows, gsem, osem):
    wid = lax.axis_index("s") * NC + lax.axis_index("c")
    idx_base = wid * SEQ_PER_W * IDX_ROWS_PER_SEQ
    row_base = wid * ROWS_PER_W

    pltpu.sync_copy(x_hbm.at[pl.ds(idx_base, SEQ_PER_W * IDX_ROWS_PER_SEQ)], idx_v)
    pltpu.sync_copy(pe_hbm, pe_v)

    def issue_gather(t, b):
        pltpu.async_copy(
            table_hbm.at[idx_v.at[2 * t]], rows[b].at[pl.ds(0, IDX_COLS)], gsem[b]
        )
        pltpu.async_copy(
            table_hbm.at[idx_v.at[2 * t + 1]],
            rows[b].at[pl.ds(IDX_COLS, IDX_COLS)],
            gsem[b],
        )

    def wait_gather(b):
        pltpu.make_async_copy(table_hbm.at[pl.ds(0, SEQ)], rows[b], gsem[b]).wait()

    def wait_outcopy(b):
        pltpu.make_async_copy(rows[b], out_hbm.at[pl.ds(0, SEQ)], osem[b]).wait()

    for b in range(LOOKAHEAD):
        issue_gather(b, b)

    def step(u, _):
        for b in range(NBUF):
            t = u * NBUF + b
            b_pre = (b + LOOKAHEAD) % NBUF

            @pl.when(t >= LOOKAHEAD)
            def _():
                wait_outcopy(b_pre)

            @pl.when(t + LOOKAHEAD < SEQ_PER_W)
            def _():
                issue_gather(t + LOOKAHEAD, b_pre)

            wait_gather(b)

            def fma_row(r, _):
                for j in range(D_MODEL // 16):
                    sl = pl.ds(j * 16, 16)
                    rows[b][r, sl] = rows[b][r, sl] * SCALE + pe_v[r, sl]
                return ()

            lax.fori_loop(0, SEQ, fma_row, ())
            pltpu.async_copy(rows[b], out_hbm.at[pl.ds(row_base + t * SEQ, SEQ)], osem[b])
        return ()

    lax.fori_loop(0, SEQ_PER_W // NBUF, step, ())
    for t in range(SEQ_PER_W - LOOKAHEAD, SEQ_PER_W):
        wait_outcopy(t % NBUF)


def kernel(x, table):
    pe = _make_pe()
    x2 = x.reshape(2 * BATCH, IDX_COLS)
    out = _sc_kernel(x2, pe, table)
    return out.reshape(BATCH, SEQ, D_MODEL)


# speedup vs baseline: 4.1985x; 1.0011x over previous
import functools
import math

import jax
import jax.numpy as jnp
import numpy as np
from jax import lax
from jax.experimental import pallas as pl
from jax.experimental.pallas import tpu as pltpu
from jax.experimental.pallas import tpu_sc as plsc

D_MODEL = 64
MAX_LEN = 5000
BATCH = 4096
SEQ = 200
SCALE = math.sqrt(D_MODEL)

NC, NS = 2, 16
NW = NC * NS
ROWS = BATCH * SEQ
SEQ_PER_W = BATCH // NW
ROWS_PER_W = SEQ_PER_W * SEQ

IDX_COLS = 100
IDX_ROWS_PER_SEQ = SEQ // IDX_COLS

NBUF = 4
LOOKAHEAD = 2


def _make_pe():
    pe = np.zeros((MAX_LEN, D_MODEL), dtype=np.float32)
    pos = np.arange(MAX_LEN, dtype=np.float32)[:, None]
    div_term = np.exp(
        np.arange(0, D_MODEL, 2, dtype=np.float32) * (-math.log(10000.0) / D_MODEL)
    )
    pe[:, 0::2] = np.sin(pos * div_term)
    pe[:, 1::2] = np.cos(pos * div_term)
    return jnp.asarray(pe[:SEQ])


_mesh = plsc.VectorSubcoreMesh(core_axis_name="c", subcore_axis_name="s")


@functools.partial(
    pl.kernel,
    out_type=jax.ShapeDtypeStruct((BATCH, SEQ, D_MODEL), jnp.float32),
    mesh=_mesh,
    scratch_types=[
        pltpu.VMEM((SEQ_PER_W * IDX_ROWS_PER_SEQ, IDX_COLS), jnp.int32),
        pltpu.VMEM((SEQ, D_MODEL), jnp.float32),
        [pltpu.VMEM((SEQ, D_MODEL), jnp.float32) for _ in range(NBUF)],
        [pltpu.SemaphoreType.DMA for _ in range(NBUF)],
        [pltpu.SemaphoreType.DMA for _ in range(NBUF)],
    ],
    compiler_params=pltpu.CompilerParams(use_tc_tiling_on_sc=False),
)
def _sc_kernel(x_hbm, pe_hbm, table_hbm, out_hbm, idx_v, pe_v, rows, gsem, osem):
    wid = lax.axis_index("s") * NC + lax.axis_index("c")
    idx_base = wid * SEQ_PER_W * IDX_ROWS_PER_SEQ
    batch_base = wid * SEQ_PER_W

    pltpu.sync_copy(x_hbm.at[pl.ds(idx_base, SEQ_PER_W * IDX_ROWS_PER_SEQ)], idx_v)
    pltpu.sync_copy(pe_hbm, pe_v)

    def issue_gather(t, b):
        pltpu.async_copy(
            table_hbm.at[idx_v.at[2 * t]], rows[b].at[pl.ds(0, IDX_COLS)], gsem[b]
        )
        pltpu.async_copy(
            table_hbm.at[idx_v.at[2 * t + 1]],
            rows[b].at[pl.ds(IDX_COLS, IDX_COLS)],
            gsem[b],
        )

    def wait_gather(b):
        pltpu.make_async_copy(table_hbm.at[pl.ds(0, SEQ)], rows[b], gsem[b]).wait()

    def wait_outcopy(b):
        pltpu.make_async_copy(rows[b], out_hbm.at[0], osem[b]).wait()

    for b in range(LOOKAHEAD):
        issue_gather(b, b)

    def step(u, _):
        for b in range(NBUF):
            t = u * NBUF + b
            b_pre = (b + LOOKAHEAD) % NBUF

            @pl.when(t >= LOOKAHEAD)
            def _():
                wait_outcopy(b_pre)

            @pl.when(t + LOOKAHEAD < SEQ_PER_W)
            def _():
                issue_gather(t + LOOKAHEAD, b_pre)

            wait_gather(b)

            def fma_row(r, _):
                for j in range(D_MODEL // 16):
                    sl = pl.ds(j * 16, 16)
                    rows[b][r, sl] = rows[b][r, sl] * SCALE + pe_v[r, sl]
                return ()

            lax.fori_loop(0, SEQ, fma_row, ())
            pltpu.async_copy(rows[b], out_hbm.at[batch_base + t], osem[b])
        return ()

    lax.fori_loop(0, SEQ_PER_W // NBUF, step, ())
    for t in range(SEQ_PER_W - LOOKAHEAD, SEQ_PER_W):
        wait_outcopy(t % NBUF)


def kernel(x, table):
    pe = _make_pe()
    x2 = x.reshape(2 * BATCH, IDX_COLS)
    return _sc_kernel(x2, pe, table)
